# baseline (speedup 1.0000x reference)
import functools

import jax
import jax.numpy as jnp
from jax import lax
from jax.experimental import pallas as pl
from jax.experimental.pallas import tpu as pltpu
from jax.experimental.pallas import tpu_sc as plsc

_JITTER = 0.1
_SAMPLE_RATE = 2048.0
_PAD = int(_JITTER * _SAMPLE_RATE)

_NC, _NS = 2, 16
_NW = _NC * _NS

_B, _C, _T = 512, 2, 8192
_R = _B * _C
_RPW = _R // _NW
_EDGE = 208
_PADB = 256
_BUF = _T + 2 * _PADB
_NBUF = 3

_mesh = plsc.VectorSubcoreMesh(
    core_axis_name="c", subcore_axis_name="s",
    num_cores=_NC, num_subcores=_NS,
)


@functools.partial(
    pl.kernel,
    out_type=jax.ShapeDtypeStruct((_B, _C, _T), jnp.float32),
    mesh=_mesh,
    scratch_types=[
        [pltpu.VMEM((_BUF,), jnp.float32) for _ in range(_NBUF)],
        [pltpu.VMEM((_T,), jnp.float32) for _ in range(_NBUF)],
        pltpu.VMEM((_RPW * 16,), jnp.int32),
        [pltpu.SemaphoreType.DMA for _ in range(_NBUF)],
        [pltpu.SemaphoreType.DMA for _ in range(_NBUF)],
    ],
    compiler_params=pltpu.CompilerParams(needs_layout_passes=False),
)
def _shift_rows(w_hbm, rv_hbm, out_hbm, in_v, out_v, rv_v, sin, sout):
    wid = lax.axis_index("s") * _NC + lax.axis_index("c")
    base = wid * _RPW
    pltpu.sync_copy(rv_hbm.at[pl.ds(base * 16, _RPW * 16)], rv_v)
    zeros = jnp.zeros((16,), jnp.float32)
    for b in range(_NBUF):
        for j in range(_EDGE // 16):
            in_v[b][pl.ds(_PADB - _EDGE + j * 16, 16)] = zeros
            in_v[b][pl.ds(_PADB + _T + j * 16, 16)] = zeros

    def in_copy(i, b):
        return pltpu.make_async_copy(
            w_hbm.at[(base + i) // _C, (base + i) % _C],
            in_v[b].at[pl.ds(_PADB, _T)], sin[b])

    _T2 = _T // 2

    def out_copy_h(i, b, h):
        return pltpu.make_async_copy(
            out_v[b].at[pl.ds(h * _T2, _T2)],
            out_hbm.at[(base + i) // _C, (base + i) % _C, pl.ds(h * _T2, _T2)],
            sout[b])

    for b in range(_NBUF):
        in_copy(b, b).start()

    for i in range(_RPW):
        b = i % _NBUF
        in_copy(i, b).wait()
        if i >= _NBUF:
            out_copy_h(i - _NBUF, b, 0).wait()
            out_copy_h(i - _NBUF, b, 1).wait()
        rv = rv_v[pl.ds(i * 16, 16)]

        @plsc.parallel_loop(0, _T2, step=16, unroll=8)
        def chunk0(t0, _rv=rv, _b=b):
            out_v[_b][pl.ds(t0, 16)] = plsc.load_gather(in_v[_b], [_rv + t0])

        out_copy_h(i, b, 0).start()

        @plsc.parallel_loop(_T2, _T, step=16, unroll=8)
        def chunk1(t0, _rv=rv, _b=b):
            out_v[_b][pl.ds(t0, 16)] = plsc.load_gather(in_v[_b], [_rv + t0])

        if i + _NBUF < _RPW:
            in_copy(i + _NBUF, b).start()
        out_copy_h(i, b, 1).start()
    for i in range(_RPW - _NBUF, _RPW):
        out_copy_h(i, i % _NBUF, 0).wait()
        out_copy_h(i, i % _NBUF, 1).wait()


def kernel(waveforms):
    B, C, T = waveforms.shape
    rkey = jax.random.key(42)
    shifts = jax.random.uniform(rkey, (B,), dtype=jnp.float32)
    shifts = 2.0 * _JITTER * shifts - _JITTER
    shifts = shifts * _SAMPLE_RATE
    shifts = shifts.astype(jnp.int32)
    src0 = jnp.repeat(_PADB + shifts, C)
    rv = src0[:, None] + jnp.arange(16, dtype=jnp.int32)[None, :]
    return _shift_rows(waveforms, rv.reshape(-1))


# speedup vs baseline: 1.0702x; 1.0702x over previous
import functools

import jax
import jax.numpy as jnp
from jax import lax
from jax.experimental import pallas as pl
from jax.experimental.pallas import tpu as pltpu
from jax.experimental.pallas import tpu_sc as plsc

_JITTER = 0.1
_SAMPLE_RATE = 2048.0
_PAD = int(_JITTER * _SAMPLE_RATE)

_NC, _NS = 2, 16
_NW = _NC * _NS

_B, _C, _T = 512, 2, 8192
_R = _B * _C
_RPW = _R // _NW
_EDGE = 208
_PADB = 256
_BUF = _T + 2 * _PADB
_NBUF = 3

_mesh = plsc.VectorSubcoreMesh(
    core_axis_name="c", subcore_axis_name="s",
    num_cores=_NC, num_subcores=_NS,
)


@functools.partial(
    pl.kernel,
    out_type=jax.ShapeDtypeStruct((_B, _C, _T), jnp.float32),
    mesh=_mesh,
    scratch_types=[
        [pltpu.VMEM((_BUF,), jnp.float32) for _ in range(_NBUF)],
        [pltpu.VMEM((_T,), jnp.float32) for _ in range(_NBUF)],
        pltpu.VMEM((_RPW * 16,), jnp.int32),
        [pltpu.SemaphoreType.DMA for _ in range(_NBUF)],
        [pltpu.SemaphoreType.DMA for _ in range(_NBUF)],
    ],
    compiler_params=pltpu.CompilerParams(needs_layout_passes=False),
)
def _shift_rows(w_hbm, rv_hbm, out_hbm, in_v, out_v, rv_v, sin, sout):
    wid = lax.axis_index("s") * _NC + lax.axis_index("c")
    base = wid * _RPW
    pltpu.sync_copy(rv_hbm.at[pl.ds(base * 16, _RPW * 16)], rv_v)
    zeros = jnp.zeros((16,), jnp.float32)
    for b in range(_NBUF):
        for j in range(_EDGE // 16):
            in_v[b][pl.ds(_PADB - _EDGE + j * 16, 16)] = zeros
            in_v[b][pl.ds(_PADB + _T + j * 16, 16)] = zeros

    def in_copy(i, b):
        return pltpu.make_async_copy(
            w_hbm.at[(base + i) // _C, (base + i) % _C],
            in_v[b].at[pl.ds(_PADB, _T)], sin[b])

    def out_copy(i, b):
        return pltpu.make_async_copy(
            out_v[b], out_hbm.at[(base + i) // _C, (base + i) % _C], sout[b])

    for b in range(_NBUF):
        in_copy(b, b).start()

    for i in range(_RPW):
        b = i % _NBUF
        in_copy(i, b).wait()
        if i >= _NBUF:
            out_copy(i - _NBUF, b).wait()
        rv = rv_v[pl.ds(i * 16, 16)]

        @plsc.parallel_loop(0, _T, step=16, unroll=8)
        def chunk(t0, _rv=rv, _b=b):
            out_v[_b][pl.ds(t0, 16)] = plsc.load_gather(in_v[_b], [_rv + t0])

        if i + _NBUF < _RPW:
            in_copy(i + _NBUF, b).start()
        out_copy(i, b).start()
    for i in range(_RPW - _NBUF, _RPW):
        out_copy(i, i % _NBUF).wait()


def kernel(waveforms):
    B, C, T = waveforms.shape
    rkey = jax.random.key(42)
    shifts = jax.random.uniform(rkey, (B,), dtype=jnp.float32)
    shifts = 2.0 * _JITTER * shifts - _JITTER
    shifts = shifts * _SAMPLE_RATE
    shifts = shifts.astype(jnp.int32)
    src0 = jnp.repeat(_PADB + shifts, C)
    rv = src0[:, None] + jnp.arange(16, dtype=jnp.int32)[None, :]
    return _shift_rows(waveforms, rv.reshape(-1))
